# baseline (speedup 1.0000x reference)
import functools

import jax
import jax.numpy as jnp
from jax import lax
from jax.experimental import pallas as pl
from jax.experimental.pallas import tpu as pltpu
from jax.experimental.pallas import tpu_sc as plsc

_NC = 2
_NS = 16
_L = 16


@functools.lru_cache(maxsize=None)
def _make_sc_kernel(n, c):
    nw = _NC * _NS
    bpw = n // nw
    nv = bpw // _L
    nrow = c // 8 * (n // 128) * 8
    itile = n // 128
    mesh = plsc.VectorSubcoreMesh(core_axis_name="c", subcore_axis_name="s")

    @functools.partial(
        pl.kernel,
        mesh=mesh,
        out_type=jax.ShapeDtypeStruct((_NC, _L), jnp.float32),
        compiler_params=pltpu.CompilerParams(needs_layout_passes=False),
        scratch_types=[
            pltpu.VMEM((bpw,), jnp.int32),
            pltpu.VMEM((bpw,), jnp.int32),
            pltpu.VMEM((bpw,), jnp.float32),
            pltpu.VMEM((bpw, 128), jnp.float32),
            pltpu.VMEM((_L,), jnp.float32),
            pltpu.VMEM((_L,), jnp.int32),
            pltpu.VMEM((_L,), jnp.float32),
            pltpu.VMEM((_L,), jnp.float32),
            pltpu.VMEM_SHARED((_L,), jnp.float32),
            pltpu.SemaphoreType.DMA,
        ],
    )
    def sc_kernel(prob_hbm, tgt_hbm, rw_hbm, out_hbm,
                  tgt_v, ridx_v, rw_v, val_v, part_v, lidx_v, red_v,
                  out_v, shared, sem):
        cid = lax.axis_index("c")
        sid = lax.axis_index("s")
        wid = sid * _NC + cid
        base = wid * bpw

        prob_rows = prob_hbm.reshape(nrow, 128)

        pltpu.sync_copy(tgt_hbm.at[pl.ds(base, bpw)], tgt_v)
        pltpu.sync_copy(rw_hbm.at[pl.ds(base, bpw)], rw_v)

        lanes = lax.iota(jnp.int32, 16)
        for j in range(nv):
            ivec = base + j * _L + lanes
            t = tgt_v[pl.ds(j * _L, _L)]
            r = (lax.shift_right_logical(t, 3) * (itile * 8)
                 + lax.shift_right_logical(ivec, 7) * 8
                 + lax.bitwise_and(t, 7))
            ridx_v[pl.ds(j * _L, _L)] = r

        pltpu.async_copy(prob_rows.at[ridx_v], val_v, sem).wait()

        part = jnp.zeros((_L,), dtype=jnp.float32)
        for j in range(nv):
            ivec = base + j * _L + lanes
            sel = plsc.load_gather(
                val_v, [j * _L + lanes, lax.bitwise_and(ivec, 127)])
            part = part + sel * rw_v[pl.ds(j * _L, _L)]
        part_v[...] = part
        lidx_v[...] = lanes

        @pl.when(sid == 0)
        def _():
            red_v[...] = jnp.zeros((_L,), dtype=jnp.float32)
            pltpu.sync_copy(red_v, shared)

        plsc.subcore_barrier()
        pltpu.sync_copy(part_v, shared.at[lidx_v], add=True)
        plsc.subcore_barrier()

        @pl.when(sid == 0)
        def _():
            pltpu.sync_copy(shared, red_v)
            out_v[...] = -red_v[...]
            pltpu.sync_copy(out_v, out_hbm.at[cid])

    return sc_kernel


def kernel(prob, target, reward):
    n, c = prob.shape
    phys = (jnp.transpose(prob)
            .reshape(c // 8, 8, n // 128, 128)
            .transpose(0, 2, 1, 3))
    out = _make_sc_kernel(n, c)(phys, target, reward)
    return jnp.sum(out)


# speedup vs baseline: 1.0633x; 1.0633x over previous
import functools

import jax
import jax.numpy as jnp
from jax import lax
from jax.experimental import pallas as pl
from jax.experimental.pallas import tpu as pltpu
from jax.experimental.pallas import tpu_sc as plsc

_NC = 1
_NS = 16
_L = 16


@functools.lru_cache(maxsize=None)
def _make_sc_kernel(n, c):
    nw = _NC * _NS
    bpw = n // nw
    nv = bpw // _L
    nrow = c // 8 * (n // 128) * 8
    itile = n // 128
    mesh = plsc.VectorSubcoreMesh(core_axis_name="c", subcore_axis_name="s", num_cores=1)

    @functools.partial(
        pl.kernel,
        mesh=mesh,
        out_type=jax.ShapeDtypeStruct((_NC, _L), jnp.float32),
        compiler_params=pltpu.CompilerParams(needs_layout_passes=False),
        scratch_types=[
            pltpu.VMEM((bpw,), jnp.int32),
            pltpu.VMEM((bpw,), jnp.int32),
            pltpu.VMEM((bpw,), jnp.float32),
            pltpu.VMEM((bpw, 128), jnp.float32),
            pltpu.VMEM((_L,), jnp.float32),
            pltpu.VMEM((_L,), jnp.int32),
            pltpu.VMEM((_L,), jnp.float32),
            pltpu.VMEM((_L,), jnp.float32),
            pltpu.VMEM_SHARED((_L,), jnp.float32),
            pltpu.SemaphoreType.DMA,
        ],
    )
    def sc_kernel(prob_hbm, tgt_hbm, rw_hbm, out_hbm,
                  tgt_v, ridx_v, rw_v, val_v, part_v, lidx_v, red_v,
                  out_v, shared, sem):
        cid = lax.axis_index("c")
        sid = lax.axis_index("s")
        wid = sid * _NC + cid
        base = wid * bpw

        prob_rows = prob_hbm.reshape(nrow, 128)

        pltpu.sync_copy(tgt_hbm.at[pl.ds(base, bpw)], tgt_v)
        pltpu.sync_copy(rw_hbm.at[pl.ds(base, bpw)], rw_v)

        lanes = lax.iota(jnp.int32, 16)
        for j in range(nv):
            ivec = base + j * _L + lanes
            t = tgt_v[pl.ds(j * _L, _L)]
            r = (lax.shift_right_logical(t, 3) * (itile * 8)
                 + lax.shift_right_logical(ivec, 7) * 8
                 + lax.bitwise_and(t, 7))
            ridx_v[pl.ds(j * _L, _L)] = r

        pltpu.async_copy(prob_rows.at[ridx_v], val_v, sem).wait()

        part = jnp.zeros((_L,), dtype=jnp.float32)
        for j in range(nv):
            ivec = base + j * _L + lanes
            sel = plsc.load_gather(
                val_v, [j * _L + lanes, lax.bitwise_and(ivec, 127)])
            part = part + sel * rw_v[pl.ds(j * _L, _L)]
        part_v[...] = part
        lidx_v[...] = lanes

        @pl.when(sid == 0)
        def _():
            red_v[...] = jnp.zeros((_L,), dtype=jnp.float32)
            pltpu.sync_copy(red_v, shared)

        plsc.subcore_barrier()
        pltpu.sync_copy(part_v, shared.at[lidx_v], add=True)
        plsc.subcore_barrier()

        @pl.when(sid == 0)
        def _():
            pltpu.sync_copy(shared, red_v)
            out_v[...] = -red_v[...]
            pltpu.sync_copy(out_v, out_hbm.at[cid])

    return sc_kernel


def kernel(prob, target, reward):
    n, c = prob.shape
    phys = (jnp.transpose(prob)
            .reshape(c // 8, 8, n // 128, 128)
            .transpose(0, 2, 1, 3))
    out = _make_sc_kernel(n, c)(phys, target, reward)
    return jnp.sum(out)


# speedup vs baseline: 1.1434x; 1.0754x over previous
import functools

import jax
import jax.numpy as jnp
from jax import lax
from jax.experimental import pallas as pl
from jax.experimental.pallas import tpu as pltpu
from jax.experimental.pallas import tpu_sc as plsc

_NS = 16
_L = 16


@functools.lru_cache(maxsize=None)
def _make_sc_kernel(n, c):
    bpw = n // _NS
    nv = bpw // _L
    nrow = c // 8 * (n // 128) * 8
    itile = n // 128
    mesh = plsc.VectorSubcoreMesh(
        core_axis_name="c", subcore_axis_name="s", num_cores=1)

    @functools.partial(
        pl.kernel,
        mesh=mesh,
        out_type=jax.ShapeDtypeStruct((1, _L), jnp.float32),
        compiler_params=pltpu.CompilerParams(needs_layout_passes=False),
        scratch_types=[
            pltpu.VMEM((bpw,), jnp.int32),
            pltpu.VMEM((bpw,), jnp.int32),
            pltpu.VMEM((bpw,), jnp.float32),
            pltpu.VMEM((bpw, 128), jnp.float32),
            pltpu.VMEM((_L,), jnp.float32),
            pltpu.VMEM((_L,), jnp.int32),
            pltpu.VMEM((_L,), jnp.float32),
            pltpu.VMEM((_L,), jnp.float32),
            pltpu.VMEM_SHARED((_L,), jnp.float32),
            pltpu.SemaphoreType.DMA,
        ],
    )
    def sc_kernel(prob_hbm, tgt_hbm, rw_hbm, out_hbm,
                  tgt_v, ridx_v, rw_v, val_v, part_v, lidx_v, red_v, out_v,
                  shared, sem):
        sid = lax.axis_index("s")
        base = sid * bpw

        prob_rows = prob_hbm.reshape(nrow, 128)

        cp_t = pltpu.async_copy(tgt_hbm.at[pl.ds(base, bpw)], tgt_v, sem)
        cp_r = pltpu.async_copy(rw_hbm.at[pl.ds(base, bpw)], rw_v, sem)
        cp_t.wait()
        cp_r.wait()

        lanes = lax.iota(jnp.int32, 16)
        for j in range(nv):
            ivec = base + j * _L + lanes
            t = tgt_v[pl.ds(j * _L, _L)]
            r = (lax.shift_right_logical(t, 3) * (itile * 8)
                 + lax.shift_right_logical(ivec, 7) * 8
                 + lax.bitwise_and(t, 7))
            ridx_v[pl.ds(j * _L, _L)] = r

        pltpu.async_copy(prob_rows.at[ridx_v], val_v, sem).wait()

        part = jnp.zeros((_L,), dtype=jnp.float32)
        for j in range(nv):
            ivec = base + j * _L + lanes
            sel = plsc.load_gather(
                val_v, [j * _L + lanes, lax.bitwise_and(ivec, 127)])
            part = part + sel * rw_v[pl.ds(j * _L, _L)]
        part_v[...] = part

        lidx_v[...] = lanes

        @pl.when(sid == 0)
        def _():
            red_v[...] = jnp.zeros((_L,), dtype=jnp.float32)
            pltpu.sync_copy(red_v, shared)

        plsc.subcore_barrier()
        pltpu.sync_copy(part_v, shared.at[lidx_v], add=True)
        plsc.subcore_barrier()

        @pl.when(sid == 0)
        def _():
            pltpu.sync_copy(shared, red_v)
            tot = red_v[...]
            for sh in (8, 4, 2, 1):
                _, partner = plsc.sort_key_val(
                    lax.bitwise_xor(lanes, sh), tot)
                tot = tot + partner
            out_v[...] = -tot
            pltpu.sync_copy(out_v, out_hbm.at[0])

    return sc_kernel


def kernel(prob, target, reward):
    n, c = prob.shape
    phys = (jnp.transpose(prob)
            .reshape(c // 8, 8, n // 128, 128)
            .transpose(0, 2, 1, 3))
    out = _make_sc_kernel(n, c)(phys, target, reward)
    return out[0, 0]


# speedup vs baseline: 1.1447x; 1.0011x over previous
import functools

import jax
import jax.numpy as jnp
from jax import lax
from jax.experimental import pallas as pl
from jax.experimental.pallas import tpu as pltpu
from jax.experimental.pallas import tpu_sc as plsc

_NS = 16
_L = 16


@functools.lru_cache(maxsize=None)
def _make_sc_kernel(n, c):
    bpw = n // _NS
    nv = bpw // _L
    nrow = c // 8 * (n // 128) * 8
    itile = n // 128
    mesh = plsc.VectorSubcoreMesh(
        core_axis_name="c", subcore_axis_name="s", num_cores=1)

    @functools.partial(
        pl.kernel,
        mesh=mesh,
        out_type=jax.ShapeDtypeStruct((1, _L), jnp.float32),
        compiler_params=pltpu.CompilerParams(
            needs_layout_passes=False,
            skip_device_barrier=True,
            disable_bounds_checks=True,
            disable_semaphore_checks=True,
        ),
        scratch_types=[
            pltpu.VMEM((bpw,), jnp.int32),
            pltpu.VMEM((bpw,), jnp.int32),
            pltpu.VMEM((bpw,), jnp.float32),
            pltpu.VMEM((bpw, 128), jnp.float32),
            pltpu.VMEM((_L,), jnp.float32),
            pltpu.VMEM((_L,), jnp.int32),
            pltpu.VMEM((_L,), jnp.float32),
            pltpu.VMEM((_L,), jnp.float32),
            pltpu.VMEM_SHARED((_L,), jnp.float32),
            pltpu.SemaphoreType.DMA,
        ],
    )
    def sc_kernel(prob_hbm, tgt_hbm, rw_hbm, out_hbm,
                  tgt_v, ridx_v, rw_v, val_v, part_v, lidx_v, red_v, out_v,
                  shared, sem):
        sid = lax.axis_index("s")
        base = sid * bpw

        prob_rows = prob_hbm.reshape(nrow, 128)

        cp_t = pltpu.async_copy(tgt_hbm.at[pl.ds(base, bpw)], tgt_v, sem)
        cp_r = pltpu.async_copy(rw_hbm.at[pl.ds(base, bpw)], rw_v, sem)
        cp_t.wait()
        cp_r.wait()

        lanes = lax.iota(jnp.int32, 16)
        for j in range(nv):
            ivec = base + j * _L + lanes
            t = tgt_v[pl.ds(j * _L, _L)]
            r = (lax.shift_right_logical(t, 3) * (itile * 8)
                 + lax.shift_right_logical(ivec, 7) * 8
                 + lax.bitwise_and(t, 7))
            ridx_v[pl.ds(j * _L, _L)] = r

        pltpu.async_copy(prob_rows.at[ridx_v], val_v, sem).wait()

        part = jnp.zeros((_L,), dtype=jnp.float32)
        for j in range(nv):
            ivec = base + j * _L + lanes
            sel = plsc.load_gather(
                val_v, [j * _L + lanes, lax.bitwise_and(ivec, 127)])
            part = part + sel * rw_v[pl.ds(j * _L, _L)]
        part_v[...] = part

        lidx_v[...] = lanes

        @pl.when(sid == 0)
        def _():
            red_v[...] = jnp.zeros((_L,), dtype=jnp.float32)
            pltpu.sync_copy(red_v, shared)

        plsc.subcore_barrier()
        pltpu.sync_copy(part_v, shared.at[lidx_v], add=True)
        plsc.subcore_barrier()

        @pl.when(sid == 0)
        def _():
            pltpu.sync_copy(shared, red_v)
            tot = red_v[...]
            for sh in (8, 4, 2, 1):
                _, partner = plsc.sort_key_val(
                    lax.bitwise_xor(lanes, sh), tot)
                tot = tot + partner
            out_v[...] = -tot
            pltpu.sync_copy(out_v, out_hbm.at[0])

    return sc_kernel


def kernel(prob, target, reward):
    n, c = prob.shape
    phys = (jnp.transpose(prob)
            .reshape(c // 8, 8, n // 128, 128)
            .transpose(0, 2, 1, 3))
    out = _make_sc_kernel(n, c)(phys, target, reward)
    return out[0, 0]


# speedup vs baseline: 1.1508x; 1.0054x over previous
import functools

import jax
import jax.numpy as jnp
from jax import lax
from jax.experimental import pallas as pl
from jax.experimental.pallas import tpu as pltpu
from jax.experimental.pallas import tpu_sc as plsc

_NS = 16
_L = 16


@functools.lru_cache(maxsize=None)
def _make_sc_kernel(n, c):
    bpw = n // _NS
    nv = bpw // _L
    nrow = c // 8 * (n // 128) * 8
    itile = n // 128
    mesh = plsc.VectorSubcoreMesh(
        core_axis_name="c", subcore_axis_name="s", num_cores=1)

    @functools.partial(
        pl.kernel,
        mesh=mesh,
        out_type=jax.ShapeDtypeStruct((1, _L), jnp.float32),
        compiler_params=pltpu.CompilerParams(
            needs_layout_passes=False,
            skip_device_barrier=True,
            disable_bounds_checks=True,
            disable_semaphore_checks=True,
        ),
        scratch_types=[
            pltpu.VMEM((bpw,), jnp.int32),
            pltpu.VMEM((bpw,), jnp.int32),
            pltpu.VMEM((bpw,), jnp.float32),
            pltpu.VMEM((bpw, 128), jnp.float32),
            pltpu.VMEM((_L,), jnp.float32),
            pltpu.VMEM((_L,), jnp.int32),
            pltpu.VMEM((_L,), jnp.float32),
            pltpu.VMEM((_L,), jnp.float32),
            pltpu.VMEM_SHARED((_L,), jnp.float32),
            pltpu.SemaphoreType.DMA,
        ],
    )
    def sc_kernel(prob_hbm, tgt_hbm, rw_hbm, out_hbm,
                  tgt_v, ridx_v, rw_v, val_v, part_v, lidx_v, red_v, out_v,
                  shared, sem):
        sid = lax.axis_index("s")
        base = sid * bpw

        prob_rows = prob_hbm.reshape(nrow, 128)

        cp_t = pltpu.async_copy(tgt_hbm.at[pl.ds(base, bpw)], tgt_v, sem)
        cp_r = pltpu.async_copy(rw_hbm.at[pl.ds(base, bpw)], rw_v, sem)

        lanes = lax.iota(jnp.int32, 16)
        lidx_v[...] = lanes

        @pl.when(sid == 0)
        def _():
            red_v[...] = jnp.zeros((_L,), dtype=jnp.float32)
            pltpu.sync_copy(red_v, shared)

        plsc.subcore_barrier()

        cp_t.wait()
        cp_r.wait()

        for j in range(nv):
            ivec = base + j * _L + lanes
            t = tgt_v[pl.ds(j * _L, _L)]
            r = (lax.shift_right_logical(t, 3) * (itile * 8)
                 + lax.shift_right_logical(ivec, 7) * 8
                 + lax.bitwise_and(t, 7))
            ridx_v[pl.ds(j * _L, _L)] = r

        pltpu.async_copy(prob_rows.at[ridx_v], val_v, sem).wait()

        part = jnp.zeros((_L,), dtype=jnp.float32)
        for j in range(nv):
            ivec = base + j * _L + lanes
            sel = plsc.load_gather(
                val_v, [j * _L + lanes, lax.bitwise_and(ivec, 127)])
            part = part + sel * rw_v[pl.ds(j * _L, _L)]
        part_v[...] = part

        pltpu.sync_copy(part_v, shared.at[lidx_v], add=True)
        plsc.subcore_barrier()

        @pl.when(sid == 0)
        def _():
            pltpu.sync_copy(shared, red_v)
            tot = red_v[...]
            for sh in (8, 4, 2, 1):
                _, partner = plsc.sort_key_val(
                    lax.bitwise_xor(lanes, sh), tot)
                tot = tot + partner
            out_v[...] = -tot
            pltpu.sync_copy(out_v, out_hbm.at[0])

    return sc_kernel


def kernel(prob, target, reward):
    n, c = prob.shape
    phys = (jnp.transpose(prob)
            .reshape(c // 8, 8, n // 128, 128)
            .transpose(0, 2, 1, 3))
    out = _make_sc_kernel(n, c)(phys, target, reward)
    return out[0, 0]
